# baseline (speedup 1.0000x reference)
import functools

import jax
import jax.numpy as jnp
from jax import lax
from jax.experimental import pallas as pl
from jax.experimental.pallas import tpu as pltpu
from jax.experimental.pallas import tpu_sc as plsc

_NC, _NS, _L = 2, 16, 16
_NW = _NC * _NS
_K = 8
_INV_R2 = 1.0 / (0.01 * 0.01)


def _sc_render(idx2, d2, feats):
    nrow, wk = idx2.shape
    w = wk // _K
    w3 = w * 3
    rows_per_w = nrow // _NW
    nchunk = rows_per_w
    ngrp = w // _L
    mesh = plsc.VectorSubcoreMesh(core_axis_name="c", subcore_axis_name="s")

    @functools.partial(
        pl.kernel,
        out_type=jax.ShapeDtypeStruct((nrow * w3,), jnp.float32),
        mesh=mesh,
        compiler_params=pltpu.CompilerParams(
            needs_layout_passes=False, use_tc_tiling_on_sc=False
        ),
        scratch_types=[
            pltpu.VMEM((wk,), jnp.int32),
            pltpu.VMEM((wk,), jnp.int32),
            pltpu.VMEM((wk,), jnp.float32),
            pltpu.VMEM((wk,), jnp.float32),
            pltpu.VMEM((wk, 8), jnp.float32),
            pltpu.VMEM((wk, 8), jnp.float32),
            pltpu.VMEM((w3,), jnp.float32),
            pltpu.VMEM((w3,), jnp.float32),
            pltpu.SemaphoreType.DMA,
            pltpu.SemaphoreType.DMA,
            pltpu.SemaphoreType.DMA,
            pltpu.SemaphoreType.DMA,
            pltpu.SemaphoreType.DMA,
            pltpu.SemaphoreType.DMA,
            pltpu.SemaphoreType.DMA,
            pltpu.SemaphoreType.DMA,
        ],
    )
    def k(idx_hbm, d_hbm, feat_hbm, out_hbm,
          iv0, iv1, dv0, dv1, rv0, rv1, ov0, ov1,
          si0, si1, sd0, sd1, sg0, sg1, so0, so1):
        wid = lax.axis_index("s") * _NC + lax.axis_index("c")
        wrow = wid * rows_per_w
        idx_v = (iv0, iv1)
        d_v = (dv0, dv1)
        rows_v = (rv0, rv1)
        out_v = (ov0, ov1)
        sem_i = (si0, si1)
        sem_d = (sd0, sd1)
        sem_g = (sg0, sg1)
        sem_o = (so0, so1)
        iota = lax.iota(jnp.int32, _L)
        iota8 = iota * _K
        iota3 = iota * 3
        c0 = jnp.zeros((_L,), jnp.int32)
        c1 = jnp.full((_L,), 1, jnp.int32)
        c2 = jnp.full((_L,), 2, jnp.int32)

        def copyin_idx(j, s):
            pltpu.async_copy(idx_hbm.at[wrow + j], idx_v[s], sem_i[s])

        def copyin_d(j, s):
            pltpu.async_copy(d_hbm.at[wrow + j], d_v[s], sem_d[s])

        def wait_idx(j, s):
            pltpu.make_async_copy(
                idx_hbm.at[wrow + j], idx_v[s], sem_i[s]).wait()

        def wait_d(j, s):
            pltpu.make_async_copy(
                d_hbm.at[wrow + j], d_v[s], sem_d[s]).wait()

        def gather(s):
            pltpu.async_copy(feat_hbm.at[idx_v[s]], rows_v[s], sem_g[s])

        def wait_gather(s):
            pltpu.make_async_copy(
                feat_hbm.at[idx_v[s]], rows_v[s], sem_g[s]).wait()

        def copyout(j, s):
            pltpu.async_copy(out_v[s], out_hbm.at[wrow + j], sem_o[s])

        def wait_out(j, s):
            pltpu.make_async_copy(
                out_v[s], out_hbm.at[wrow + j], sem_o[s]).wait()

        def compute(s):
            dv = d_v[s]
            rv = rows_v[s]
            ov = out_v[s]

            @pl.loop(0, ngrp)
            def _group(g):
                base = g * (_L * _K)
                den = jnp.full((_L,), 1e-10, jnp.float32)
                a0 = jnp.zeros((_L,), jnp.float32)
                a1 = jnp.zeros((_L,), jnp.float32)
                a2 = jnp.zeros((_L,), jnp.float32)
                for kk in range(_K):
                    ridx = iota8 + (base + kk)
                    d = plsc.load_gather(dv, [ridx])
                    wgt = 1.0 - d * _INV_R2
                    den = den + wgt
                    f0 = plsc.load_gather(rv, [ridx, c0])
                    f1 = plsc.load_gather(rv, [ridx, c1])
                    f2 = plsc.load_gather(rv, [ridx, c2])
                    a0 = a0 + wgt * f0
                    a1 = a1 + wgt * f1
                    a2 = a2 + wgt * f2
                inv = 1.0 / den
                obase = iota3 + g * (_L * 3)
                plsc.store_scatter(ov, [obase], a0 * inv)
                plsc.store_scatter(ov, [obase + 1], a1 * inv)
                plsc.store_scatter(ov, [obase + 2], a2 * inv)

        @pl.loop(0, nchunk)
        def _chunk(j):
            e0 = (wrow + j) * wk
            o0 = (wrow + j) * w3
            pltpu.sync_copy(idx_hbm.at[pl.ds(e0, wk)], idx_v[0])
            pltpu.sync_copy(d_hbm.at[pl.ds(e0, wk)], d_v[0])
            pltpu.async_copy(feat_hbm.at[idx_v[0]], rows_v[0], sem_g[0]).wait()
            compute(0)
            pltpu.sync_copy(out_v[0], out_hbm.at[pl.ds(o0, w3)])

    return k(idx2.reshape(-1), d2.reshape(-1), feats)


def kernel(idx, dists, features_packed):
    b, h, w, kk = idx.shape
    idx2 = idx.reshape(b * h, w * kk).astype(jnp.int32)
    d2 = dists.reshape(b * h, w * kk)
    feats8 = jnp.pad(features_packed, ((0, 0), (0, 5)))
    out2 = _sc_render(idx2, d2, feats8)
    images = out2.reshape(b, h, w, 3)
    return images, idx, dists


# speedup vs baseline: 2.3141x; 2.3141x over previous
import functools

import jax
import jax.numpy as jnp
from jax import lax
from jax.experimental import pallas as pl
from jax.experimental.pallas import tpu as pltpu
from jax.experimental.pallas import tpu_sc as plsc

_NC, _NS, _L = 2, 16, 16
_NW = _NC * _NS
_K = 8
_INV_R2 = 1.0 / (0.01 * 0.01)


def _sc_render(idx2, d2, feats):
    nrow, wk = idx2.shape
    w = wk // _K
    w3 = w * 3
    rows_per_w = nrow // _NW
    nchunk = rows_per_w
    ngrp = w // _L
    mesh = plsc.VectorSubcoreMesh(core_axis_name="c", subcore_axis_name="s")

    @functools.partial(
        pl.kernel,
        out_type=jax.ShapeDtypeStruct((nrow, w3), jnp.float32),
        mesh=mesh,
        compiler_params=pltpu.CompilerParams(
            needs_layout_passes=False, use_tc_tiling_on_sc=False
        ),
        scratch_types=[
            pltpu.VMEM((wk,), jnp.int32),
            pltpu.VMEM((wk,), jnp.int32),
            pltpu.VMEM((wk,), jnp.float32),
            pltpu.VMEM((wk,), jnp.float32),
            pltpu.VMEM((wk, 8), jnp.float32),
            pltpu.VMEM((wk, 8), jnp.float32),
            pltpu.VMEM((w3,), jnp.float32),
            pltpu.VMEM((w3,), jnp.float32),
            pltpu.SemaphoreType.DMA,
            pltpu.SemaphoreType.DMA,
            pltpu.SemaphoreType.DMA,
            pltpu.SemaphoreType.DMA,
            pltpu.SemaphoreType.DMA,
            pltpu.SemaphoreType.DMA,
            pltpu.SemaphoreType.DMA,
            pltpu.SemaphoreType.DMA,
        ],
    )
    def k(idx_hbm, d_hbm, feat_hbm, out_hbm,
          iv0, iv1, dv0, dv1, rv0, rv1, ov0, ov1,
          si0, si1, sd0, sd1, sg0, sg1, so0, so1):
        wid = lax.axis_index("s") * _NC + lax.axis_index("c")
        wrow = wid * rows_per_w
        idx_v = (iv0, iv1)
        d_v = (dv0, dv1)
        rows_v = (rv0, rv1)
        out_v = (ov0, ov1)
        sem_i = (si0, si1)
        sem_d = (sd0, sd1)
        sem_g = (sg0, sg1)
        sem_o = (so0, so1)
        iota = lax.iota(jnp.int32, _L)
        iota8 = iota * _K
        iota3 = iota * 3
        c0 = jnp.zeros((_L,), jnp.int32)
        c1 = jnp.full((_L,), 1, jnp.int32)
        c2 = jnp.full((_L,), 2, jnp.int32)

        def copyin_idx(j, s):
            pltpu.async_copy(idx_hbm.at[wrow + j], idx_v[s], sem_i[s])

        def copyin_d(j, s):
            pltpu.async_copy(d_hbm.at[wrow + j], d_v[s], sem_d[s])

        def wait_idx(j, s):
            pltpu.make_async_copy(
                idx_hbm.at[wrow + j], idx_v[s], sem_i[s]).wait()

        def wait_d(j, s):
            pltpu.make_async_copy(
                d_hbm.at[wrow + j], d_v[s], sem_d[s]).wait()

        def gather(s):
            pltpu.async_copy(feat_hbm.at[idx_v[s]], rows_v[s], sem_g[s])

        def wait_gather(s):
            pltpu.make_async_copy(
                feat_hbm.at[idx_v[s]], rows_v[s], sem_g[s]).wait()

        def copyout(j, s):
            pltpu.async_copy(out_v[s], out_hbm.at[wrow + j], sem_o[s])

        def wait_out(j, s):
            pltpu.make_async_copy(
                out_v[s], out_hbm.at[wrow + j], sem_o[s]).wait()

        def compute(s):
            dv = d_v[s]
            rv = rows_v[s]
            ov = out_v[s]

            @pl.loop(0, ngrp)
            def _group(g):
                base = g * (_L * _K)
                den = jnp.full((_L,), 1e-10, jnp.float32)
                a0 = jnp.zeros((_L,), jnp.float32)
                a1 = jnp.zeros((_L,), jnp.float32)
                a2 = jnp.zeros((_L,), jnp.float32)
                for kk in range(_K):
                    ridx = iota8 + (base + kk)
                    d = plsc.load_gather(dv, [ridx])
                    wgt = 1.0 - d * _INV_R2
                    den = den + wgt
                    f0 = plsc.load_gather(rv, [ridx, c0])
                    f1 = plsc.load_gather(rv, [ridx, c1])
                    f2 = plsc.load_gather(rv, [ridx, c2])
                    a0 = a0 + wgt * f0
                    a1 = a1 + wgt * f1
                    a2 = a2 + wgt * f2
                inv = 1.0 / den
                obase = iota3 + g * (_L * 3)
                plsc.store_scatter(ov, [obase], a0 * inv)
                plsc.store_scatter(ov, [obase + 1], a1 * inv)
                plsc.store_scatter(ov, [obase + 2], a2 * inv)

        pltpu.sync_copy(idx_hbm.at[wrow], idx_v[0])
        pltpu.sync_copy(d_hbm.at[wrow], d_v[0])
        gather(0)

        @pl.loop(0, nchunk, step=2)
        def _pair(ci):
            for b in range(2):
                j = ci + b
                s = b

                @pl.when(j + 1 <= nchunk - 1)
                def _():
                    pltpu.sync_copy(idx_hbm.at[wrow + j + 1], idx_v[1 - s])
                    pltpu.sync_copy(d_hbm.at[wrow + j + 1], d_v[1 - s])
                    gather(1 - s)

                wait_gather(s)
                compute(s)
                pltpu.sync_copy(out_v[s], out_hbm.at[wrow + j])

    return k(idx2, d2, feats)


def kernel(idx, dists, features_packed):
    b, h, w, kk = idx.shape
    idx2 = idx.reshape(b * h, w * kk).astype(jnp.int32)
    d2 = dists.reshape(b * h, w * kk)
    feats8 = jnp.pad(features_packed, ((0, 0), (0, 5)))
    out2 = _sc_render(idx2, d2, feats8)
    images = out2.reshape(b, h, w, 3)
    return images, idx, dists


# speedup vs baseline: 4.0200x; 1.7372x over previous
import functools

import jax
import jax.numpy as jnp
from jax import lax
from jax.experimental import pallas as pl
from jax.experimental.pallas import tpu as pltpu
from jax.experimental.pallas import tpu_sc as plsc

_NC, _NS, _L = 2, 16, 16
_K = 8
_INV_R2 = 1.0 / (0.01 * 0.01)
_HI = -65536


def _sc_render(idx2, d2, t01, t2p):
    nrow, wk = idx2.shape
    w = wk // _K
    npoints = t01.shape[0]
    rows_per_tile = nrow // _NS
    ngrp = w // _L
    mesh = plsc.VectorSubcoreMesh(core_axis_name="c", subcore_axis_name="s")

    @functools.partial(
        pl.kernel,
        out_type=(
            jax.ShapeDtypeStruct((nrow, w * 2), jnp.float32),
            jax.ShapeDtypeStruct((nrow, w), jnp.float32),
        ),
        mesh=mesh,
        compiler_params=pltpu.CompilerParams(
            needs_layout_passes=False, use_tc_tiling_on_sc=False
        ),
        scratch_types=[
            pltpu.VMEM((npoints,), jnp.int32),
            pltpu.VMEM((wk,), jnp.int32),
            pltpu.VMEM((wk,), jnp.int32),
            pltpu.VMEM((wk,), jnp.float32),
            pltpu.VMEM((wk,), jnp.float32),
            pltpu.VMEM((w * 2,), jnp.float32),
            pltpu.VMEM((w * 2,), jnp.float32),
            pltpu.VMEM((w,), jnp.float32),
            pltpu.VMEM((w,), jnp.float32),
            pltpu.SemaphoreType.DMA,
            pltpu.SemaphoreType.DMA,
            pltpu.SemaphoreType.DMA,
            pltpu.SemaphoreType.DMA,
        ],
    )
    def k(idx_hbm, d_hbm, t01_hbm, t2p_hbm, o01_hbm, o2_hbm,
          tbl, iv0, iv1, dv0, dv1, a0_, a1_, b0_, b1_,
          si0, si1, sd0, sd1):
        cid = lax.axis_index("c")
        sid = lax.axis_index("s")
        wrow = sid * rows_per_tile
        idx_v = (iv0, iv1)
        d_v = (dv0, dv1)
        o01_v = (a0_, a1_)
        o2_v = (b0_, b1_)
        sem_i = (si0, si1)
        sem_d = (sd0, sd1)
        iota = lax.iota(jnp.int32, _L)
        iota8 = iota * _K
        iota2 = iota * 2

        @pl.when(cid == 0)
        def _():
            pltpu.sync_copy(t01_hbm, tbl)

        @pl.when(cid == 1)
        def _():
            pltpu.sync_copy(t2p_hbm, tbl)

        def copyin_idx(j, s):
            pltpu.async_copy(idx_hbm.at[wrow + j], idx_v[s], sem_i[s])

        def copyin_d(j, s):
            pltpu.async_copy(d_hbm.at[wrow + j], d_v[s], sem_d[s])

        def wait_idx(j, s):
            pltpu.make_async_copy(
                idx_hbm.at[wrow + j], idx_v[s], sem_i[s]).wait()

        def wait_d(j, s):
            pltpu.make_async_copy(
                d_hbm.at[wrow + j], d_v[s], sem_d[s]).wait()

        def compute01(s):
            iv = idx_v[s]
            dv = d_v[s]
            ov = o01_v[s]

            @pl.loop(0, ngrp)
            def _group(g):
                base = g * (_L * _K)
                den = jnp.full((_L,), 1e-10, jnp.float32)
                a0 = jnp.zeros((_L,), jnp.float32)
                a1 = jnp.zeros((_L,), jnp.float32)
                for kk in range(_K):
                    ridx = iota8 + (base + kk)
                    pid = plsc.load_gather(iv, [ridx])
                    d = plsc.load_gather(dv, [ridx])
                    wgt = 1.0 - d * _INV_R2
                    den = den + wgt
                    wv = plsc.load_gather(tbl, [pid])
                    f0 = plsc.bitcast(wv & _HI, jnp.float32)
                    f1 = plsc.bitcast(wv << 16, jnp.float32)
                    a0 = a0 + wgt * f0
                    a1 = a1 + wgt * f1
                inv = 1.0 / den
                obase = iota2 + g * (_L * 2)
                plsc.store_scatter(ov, [obase], a0 * inv)
                plsc.store_scatter(ov, [obase + 1], a1 * inv)

        def compute2(s):
            iv = idx_v[s]
            dv = d_v[s]
            ov = o2_v[s]

            @pl.loop(0, ngrp)
            def _group(g):
                base = g * (_L * _K)
                den = jnp.full((_L,), 1e-10, jnp.float32)
                a2 = jnp.zeros((_L,), jnp.float32)
                for kk in range(_K):
                    ridx = iota8 + (base + kk)
                    pid = plsc.load_gather(iv, [ridx])
                    d = plsc.load_gather(dv, [ridx])
                    wgt = 1.0 - d * _INV_R2
                    den = den + wgt
                    wv = plsc.load_gather(tbl, [pid >> 1])
                    sh = (1 - (pid & 1)) << 4
                    f2 = plsc.bitcast((wv << sh) & _HI, jnp.float32)
                    a2 = a2 + wgt * f2
                inv = 1.0 / den
                ov_idx = iota + g * _L
                plsc.store_scatter(ov, [ov_idx], a2 * inv)

        copyin_idx(0, 0)
        copyin_d(0, 0)

        @pl.loop(0, rows_per_tile, step=2)
        def _pair(ci):
            for b in range(2):
                j = ci + b
                s = b

                @pl.when(j + 1 <= rows_per_tile - 1)
                def _():
                    copyin_idx(j + 1, 1 - s)
                    copyin_d(j + 1, 1 - s)

                wait_idx(j, s)
                wait_d(j, s)

                @pl.when(cid == 0)
                def _():
                    compute01(s)
                    pltpu.sync_copy(o01_v[s], o01_hbm.at[wrow + j])

                @pl.when(cid == 1)
                def _():
                    compute2(s)
                    pltpu.sync_copy(o2_v[s], o2_hbm.at[wrow + j])

    return k(idx2, d2, t01, t2p)


def kernel(idx, dists, features_packed):
    b, h, w, kk = idx.shape
    idx2 = idx.reshape(b * h, w * kk).astype(jnp.int32)
    d2 = dists.reshape(b * h, w * kk)
    fb = lax.bitcast_convert_type(
        features_packed.astype(jnp.bfloat16), jnp.uint16
    ).astype(jnp.int32)
    t01 = (fb[:, 0] << 16) | fb[:, 1]
    c2 = fb[:, 2]
    t2p = (c2[1::2] << 16) | c2[0::2]
    t2p_full = jnp.concatenate([t2p, jnp.zeros_like(t2p)])
    out01, out2 = _sc_render(idx2, d2, t01, t2p_full)
    images = jnp.concatenate(
        [out01.reshape(b, h, w, 2), out2.reshape(b, h, w, 1)], axis=-1
    )
    return images, idx, dists


# speedup vs baseline: 7.8451x; 1.9515x over previous
import functools

import jax
import jax.numpy as jnp
from jax import lax
from jax.experimental import pallas as pl
from jax.experimental.pallas import tpu as pltpu
from jax.experimental.pallas import tpu_sc as plsc

_NC, _NS, _L = 2, 16, 16
_K = 8
_INV_R2 = 1.0 / (0.01 * 0.01)
_HI = -65536


def _sc_render(idx2, d2, t01, t2p):
    nrow, wk = idx2.shape
    w = wk // _K
    npoints = t01.shape[0]
    rows_per_tile = nrow // _NS
    ngrp = w // _L
    mesh = plsc.VectorSubcoreMesh(core_axis_name="c", subcore_axis_name="s")

    @functools.partial(
        pl.kernel,
        out_type=(
            jax.ShapeDtypeStruct((nrow, w * 2), jnp.float32),
            jax.ShapeDtypeStruct((nrow, w), jnp.float32),
        ),
        mesh=mesh,
        compiler_params=pltpu.CompilerParams(
            needs_layout_passes=False, use_tc_tiling_on_sc=False
        ),
        scratch_types=[
            pltpu.VMEM((npoints,), jnp.int32),
            pltpu.VMEM((wk,), jnp.int32),
            pltpu.VMEM((wk,), jnp.int32),
            pltpu.VMEM((wk,), jnp.float32),
            pltpu.VMEM((wk,), jnp.float32),
            pltpu.VMEM((w * 2,), jnp.float32),
            pltpu.VMEM((w * 2,), jnp.float32),
            pltpu.VMEM((w,), jnp.float32),
            pltpu.VMEM((w,), jnp.float32),
            pltpu.SemaphoreType.DMA,
            pltpu.SemaphoreType.DMA,
            pltpu.SemaphoreType.DMA,
            pltpu.SemaphoreType.DMA,
        ],
    )
    def k(idx_hbm, d_hbm, t01_hbm, t2p_hbm, o01_hbm, o2_hbm,
          tbl, iv0, iv1, dv0, dv1, a0_, a1_, b0_, b1_,
          si0, si1, sd0, sd1):
        cid = lax.axis_index("c")
        sid = lax.axis_index("s")
        wrow = sid * rows_per_tile
        idx_v = (iv0, iv1)
        d_v = (dv0, dv1)
        o01_v = (a0_, a1_)
        o2_v = (b0_, b1_)
        sem_i = (si0, si1)
        sem_d = (sd0, sd1)
        iota = lax.iota(jnp.int32, _L)
        iota8 = iota * _K
        iota2 = iota * 2

        @pl.when(cid == 0)
        def _():
            pltpu.sync_copy(t01_hbm, tbl)

        @pl.when(cid == 1)
        def _():
            pltpu.sync_copy(t2p_hbm, tbl)

        def copyin_idx(j, s):
            pltpu.async_copy(idx_hbm.at[wrow + j], idx_v[s], sem_i[s])

        def copyin_d(j, s):
            pltpu.async_copy(d_hbm.at[wrow + j], d_v[s], sem_d[s])

        def wait_idx(j, s):
            pltpu.make_async_copy(
                idx_hbm.at[wrow + j], idx_v[s], sem_i[s]).wait()

        def wait_d(j, s):
            pltpu.make_async_copy(
                d_hbm.at[wrow + j], d_v[s], sem_d[s]).wait()

        def compute01(s):
            iv = idx_v[s]
            dv = d_v[s]
            ov = o01_v[s]

            @pl.loop(0, ngrp)
            def _group(g):
                w0 = g * _L
                den = jnp.full((_L,), 1e-10, jnp.float32)
                a0 = jnp.zeros((_L,), jnp.float32)
                a1 = jnp.zeros((_L,), jnp.float32)
                for kk in range(_K):
                    off = kk * w + w0
                    pid = iv[pl.ds(off, _L)]
                    d = dv[pl.ds(off, _L)]
                    wgt = 1.0 - d * _INV_R2
                    den = den + wgt
                    wv = plsc.load_gather(tbl, [pid])
                    f0 = plsc.bitcast(wv & _HI, jnp.float32)
                    f1 = plsc.bitcast(wv << 16, jnp.float32)
                    a0 = a0 + wgt * f0
                    a1 = a1 + wgt * f1
                inv = 1.0 / den
                ov[pl.ds(w0, _L)] = a0 * inv
                ov[pl.ds(w + w0, _L)] = a1 * inv

        def compute2(s):
            iv = idx_v[s]
            dv = d_v[s]
            ov = o2_v[s]

            @pl.loop(0, ngrp)
            def _group(g):
                w0 = g * _L
                den = jnp.full((_L,), 1e-10, jnp.float32)
                a2 = jnp.zeros((_L,), jnp.float32)
                for kk in range(_K):
                    off = kk * w + w0
                    pid = iv[pl.ds(off, _L)]
                    d = dv[pl.ds(off, _L)]
                    wgt = 1.0 - d * _INV_R2
                    den = den + wgt
                    wv = plsc.load_gather(tbl, [pid >> 1])
                    sh = (1 - (pid & 1)) << 4
                    f2 = plsc.bitcast((wv << sh) & _HI, jnp.float32)
                    a2 = a2 + wgt * f2
                inv = 1.0 / den
                ov[pl.ds(w0, _L)] = a2 * inv

        copyin_idx(0, 0)
        copyin_d(0, 0)

        @pl.loop(0, rows_per_tile, step=2)
        def _pair(ci):
            for b in range(2):
                j = ci + b
                s = b

                @pl.when(j + 1 <= rows_per_tile - 1)
                def _():
                    copyin_idx(j + 1, 1 - s)
                    copyin_d(j + 1, 1 - s)

                wait_idx(j, s)
                wait_d(j, s)

                @pl.when(cid == 0)
                def _():
                    compute01(s)
                    pltpu.sync_copy(o01_v[s], o01_hbm.at[wrow + j])

                @pl.when(cid == 1)
                def _():
                    compute2(s)
                    pltpu.sync_copy(o2_v[s], o2_hbm.at[wrow + j])

    return k(idx2, d2, t01, t2p)


def kernel(idx, dists, features_packed):
    b, h, w, kk = idx.shape
    idx2 = jnp.transpose(idx, (0, 1, 3, 2)).reshape(b * h, kk * w)
    idx2 = idx2.astype(jnp.int32)
    d2 = jnp.transpose(dists, (0, 1, 3, 2)).reshape(b * h, kk * w)
    fb = lax.bitcast_convert_type(
        features_packed.astype(jnp.bfloat16), jnp.uint16
    ).astype(jnp.int32)
    t01 = (fb[:, 0] << 16) | fb[:, 1]
    c2 = fb[:, 2]
    t2p = (c2[1::2] << 16) | c2[0::2]
    t2p_full = jnp.concatenate([t2p, jnp.zeros_like(t2p)])
    out01, out2 = _sc_render(idx2, d2, t01, t2p_full)
    planes = jnp.concatenate(
        [out01.reshape(b, h, 2, w), out2.reshape(b, h, 1, w)], axis=2
    )
    images = jnp.transpose(planes, (0, 1, 3, 2))
    return images, idx, dists
